# baseline (speedup 1.0000x reference)
import jax
import jax.numpy as jnp
from jax.experimental import pallas as pl
from jax.experimental.pallas import tpu as pltpu

_N = 1048576
_D = 2
_LANES = 128
_G = _N // _LANES
_BLK = 512


def _body(scale_ref, shift_ref, eps_ref, prob_ref, out_ref):
    e = eps_ref[...]
    sub = jax.lax.broadcasted_iota(jnp.int32, e.shape, 0)
    even = (sub & 1) == 0
    sv = jnp.where(even, scale_ref[0], scale_ref[1])
    tv = jnp.where(even, shift_ref[0], shift_ref[1])
    z = sv * e + tv
    u = z * z
    v = u + jnp.concatenate([u[1:, :], u[:1, :]], axis=0)
    s2 = jnp.where(even, v, jnp.concatenate([v[-1:, :], v[:-1, :]], axis=0))
    c = jnp.float32(-0.5 * _D * jnp.log(2.0 * jnp.pi))
    lp = c - 0.5 * s2
    p_ = jnp.exp(lp - c)
    pe = jnp.repeat(prob_ref[...], _D, axis=0)
    out_ref[...] = jnp.where(p_ > pe, z, jnp.zeros_like(z))


def kernel(eps, prob, prop_scale, prop_shift):
    eps_flat = (eps.reshape(_G, _LANES, _D)
                .transpose(0, 2, 1)
                .reshape(_G * _D, _LANES))
    prob_t = prob.reshape(_G, _LANES)
    out = pl.pallas_call(
        _body,
        grid=(_G // _BLK,),
        in_specs=[
            pl.BlockSpec(memory_space=pltpu.SMEM),
            pl.BlockSpec(memory_space=pltpu.SMEM),
            pl.BlockSpec((_BLK * _D, _LANES), lambda i: (i, 0)),
            pl.BlockSpec((_BLK, _LANES), lambda i: (i, 0)),
        ],
        out_specs=pl.BlockSpec((_BLK * _D, _LANES), lambda i: (i, 0)),
        out_shape=jax.ShapeDtypeStruct((_G * _D, _LANES), jnp.float32),
    )(prop_scale, prop_shift, eps_flat, prob_t)
    return (out.reshape(_G, _D, _LANES)
            .transpose(0, 2, 1)
            .reshape(_N, _D))


# speedup vs baseline: 1.3394x; 1.3394x over previous
import jax
import jax.numpy as jnp
from jax.experimental import pallas as pl
from jax.experimental.pallas import tpu as pltpu

_N = 1048576
_D = 2
_LANES = 128
_G = _N // _LANES
_BLK = 2048


def _body(scale_ref, shift_ref, eps_ref, prob_ref, out_ref):
    e = eps_ref[...]
    sub = jax.lax.broadcasted_iota(jnp.int32, e.shape, 0)
    even = (sub & 1) == 0
    sv = jnp.where(even, scale_ref[0], scale_ref[1])
    tv = jnp.where(even, shift_ref[0], shift_ref[1])
    z = sv * e + tv
    u = z * z
    v = u + jnp.concatenate([u[1:, :], u[:1, :]], axis=0)
    s2 = jnp.where(even, v, jnp.concatenate([v[-1:, :], v[:-1, :]], axis=0))
    c = jnp.float32(-0.5 * _D * jnp.log(2.0 * jnp.pi))
    lp = c - 0.5 * s2
    p_ = jnp.exp(lp - c)
    pe = jnp.repeat(prob_ref[...], _D, axis=0)
    out_ref[...] = jnp.where(p_ > pe, z, jnp.zeros_like(z))


def kernel(eps, prob, prop_scale, prop_shift):
    eps_flat = (eps.reshape(_G, _LANES, _D)
                .transpose(0, 2, 1)
                .reshape(_G * _D, _LANES))
    prob_t = prob.reshape(_G, _LANES)
    out = pl.pallas_call(
        _body,
        grid=(_G // _BLK,),
        in_specs=[
            pl.BlockSpec(memory_space=pltpu.SMEM),
            pl.BlockSpec(memory_space=pltpu.SMEM),
            pl.BlockSpec((_BLK * _D, _LANES), lambda i: (i, 0)),
            pl.BlockSpec((_BLK, _LANES), lambda i: (i, 0)),
        ],
        out_specs=pl.BlockSpec((_BLK * _D, _LANES), lambda i: (i, 0)),
        out_shape=jax.ShapeDtypeStruct((_G * _D, _LANES), jnp.float32),
    )(prop_scale, prop_shift, eps_flat, prob_t)
    return (out.reshape(_G, _D, _LANES)
            .transpose(0, 2, 1)
            .reshape(_N, _D))
